# baseline (speedup 1.0000x reference)
import functools

import jax
import jax.numpy as jnp
from jax import lax
from jax.experimental import pallas as pl
from jax.experimental.pallas import tpu as pltpu
from jax.experimental.pallas import tpu_sc as plsc

N = 10000
M = 10000
D = 128
AN = 8
AE = 16
ALPHA = 0.2

L = 16
NW = 32
NB = 10000 // L
LSTAGE = 20
assert (NW - 1) * NB // NW + LSTAGE <= NB


def _mm_score_body(x_ref, w_ref, a_ref, y_ref, s_ref, *, leaky, pack):
    y = jnp.dot(x_ref[...], w_ref[...], preferred_element_type=jnp.float32)
    if pack:
        yi = jax.lax.bitcast_convert_type(y, jnp.int32) + 0x8000
        lo = jnp.right_shift(yi[:, : D // 2], 16) & 0xFFFF
        hi = yi[:, D // 2:] & jnp.int32(-65536)
        y_ref[...] = lo | hi
    else:
        y_ref[...] = y
    u = jnp.where(y >= 0, y, ALPHA * y) if leaky else y
    s_ref[...] = jnp.dot(u, a_ref[...], preferred_element_type=jnp.float32)


def _mm_score(x, w, a_row, leaky, pack=False):
    R = x.shape[0]
    B = 2000
    YW = D // 2 if pack else D
    return pl.pallas_call(
        functools.partial(_mm_score_body, leaky=leaky, pack=pack),
        grid=(R // B,),
        in_specs=[
            pl.BlockSpec((B, D), lambda i: (i, 0)),
            pl.BlockSpec((D, D), lambda i: (0, 0)),
            pl.BlockSpec((D, 1), lambda i: (0, 0)),
        ],
        out_specs=[
            pl.BlockSpec((B, YW), lambda i: (i, 0)),
            pl.BlockSpec((B, 1), lambda i: (i, 0)),
        ],
        out_shape=[
            jax.ShapeDtypeStruct((R, YW), jnp.int32 if pack else jnp.float32),
            jax.ShapeDtypeStruct((R, 1), jnp.float32),
        ],
    )(x, w, a_row)


def _agg_body(table_hbm, score_hbm, idx_hbm, out_hbm,
              score_v, el_v, rows, w_v, o_v, sems, semo, sem_s,
              *, A, T, packed_bf16):
    IPB = L * A
    c = lax.axis_index("c")
    s = lax.axis_index("s")
    wid = s * 2 + c
    b0 = wid * NB // NW
    b1 = (wid + 1) * NB // NW
    nb = b1 - b0

    score_cp = pltpu.make_async_copy(score_hbm, score_v, sem_s)
    score_cp.start()
    pltpu.sync_copy(idx_hbm.at[pl.ds(b0 * IPB, LSTAGE * IPB)], el_v)
    lanes = lax.iota(jnp.int32, L)

    def start(i, rows, sem):
        pltpu.make_async_copy(
            table_hbm.at[el_v.at[pl.ds(i * IPB, IPB)]], rows, sem).start()

    def softmax_weights(i):
        off = i * IPB
        sjs = []
        for j in range(A):
            ij = plsc.load_gather(el_v, [off + lanes * A + j])
            sjs.append(plsc.load_gather(score_v, [ij]))
        mx = sjs[0]
        for j in range(1, A):
            mx = jnp.maximum(mx, sjs[j])
        es = [jnp.exp(sj - mx) for sj in sjs]
        z = es[0]
        for j in range(1, A):
            z = z + es[j]
        rz = 1.0 / z
        for j in range(A):
            w_v[pl.ds(j * L, L)] = es[j] * rz

    def weighted_sum(i, rows_v, o_v):
        def edge_body(e, carry2):
            acc = [jnp.zeros((L,), jnp.float32)] * (D // L)
            for j in range(A):
                wj = plsc.load_gather(
                    w_v, [jnp.broadcast_to(j * L + e, (L,))])
                r = e * A + j
                if not packed_bf16:
                    for ch in range(D // L):
                        acc[ch] = acc[ch] + wj * rows_v[r, pl.ds(ch * L, L)]
                else:
                    nh = D // (2 * L)
                    for ch in range(nh):
                        word = rows_v[r, pl.ds(ch * L, L)]
                        loc = plsc.bitcast(word << 16, jnp.float32)
                        hic = plsc.bitcast(
                            word & jnp.int32(-65536), jnp.float32)
                        acc[ch] = acc[ch] + wj * loc
                        acc[nh + ch] = acc[nh + ch] + wj * hic
            for ch in range(D // L):
                o_v[e, pl.ds(ch * L, L)] = acc[ch]
            return carry2

        lax.fori_loop(0, L, edge_body, 0)

    def out_start(i, o_v, semo):
        pltpu.make_async_copy(
            o_v, out_hbm.at[pl.ds((b0 + i) * L, L)], semo).start()

    def out_wait(i, o_v, semo):
        pltpu.make_async_copy(
            o_v, out_hbm.at[pl.ds((b0 + i) * L, L)], semo).wait()

    start(0, rows[0], sems[0])
    start(1, rows[1], sems[1])
    start(2, rows[2], sems[2])
    score_cp.wait()

    def stage(g, p, k):
        @pl.when(g + 3 < nb)
        def _():
            start(g + 3, rows[(k + 3) % 4], sems[(k + 3) % 4])

        softmax_weights(g)
        if k < 2:
            @pl.when(p > 0)
            def _():
                out_wait(g - 2, o_v[k % 2], semo[k % 2])
        else:
            out_wait(g - 2, o_v[k % 2], semo[k % 2])
        pltpu.make_async_copy(
            table_hbm.at[el_v.at[pl.ds(g * IPB, IPB)]],
            rows[k], sems[k]).wait()
        weighted_sum(g, rows[k], o_v[k % 2])
        out_start(g, o_v[k % 2], semo[k % 2])

    def quad_body(p, carry):
        g = 4 * p
        stage(g, p, 0)
        stage(g + 1, p, 1)
        stage(g + 2, p, 2)

        @pl.when(g + 3 < nb)
        def _():
            stage(g + 3, p, 3)

        return carry

    lax.fori_loop(0, LSTAGE // 4, quad_body, 0)
    out_wait(LSTAGE - 2, o_v[0], semo[0])
    out_wait(LSTAGE - 2, o_v[1], semo[1])


def _agg(table, score, idx_flat, A, packed_bf16=False):
    O = idx_flat.shape[0] // A
    T = table.shape[0]
    W = table.shape[1]
    mesh = plsc.VectorSubcoreMesh(core_axis_name="c", subcore_axis_name="s")
    kfn = functools.partial(
        pl.kernel,
        out_type=jax.ShapeDtypeStruct((O, D), jnp.float32),
        mesh=mesh,
        compiler_params=pltpu.CompilerParams(
            needs_layout_passes=False,
            use_tc_tiling_on_sc=not packed_bf16),
        scratch_types=[
            pltpu.VMEM((T,), jnp.float32),
            pltpu.VMEM((LSTAGE * L * A,), jnp.int32),
            [pltpu.VMEM((L * A, W), table.dtype)
             for _ in range(4)],
            pltpu.VMEM((A * L,), jnp.float32),
            [pltpu.VMEM((L, D), jnp.float32)
             for _ in range(2)],
            [pltpu.SemaphoreType.DMA for _ in range(4)],
            [pltpu.SemaphoreType.DMA for _ in range(2)],
            pltpu.SemaphoreType.DMA,
        ],
    )
    body = functools.partial(_agg_body, A=A, T=T, packed_bf16=packed_bf16)
    return kfn(body)(table, score, idx_flat)


def _agg_jnp(table, score, idx_flat, A):
    O = idx_flat.shape[0] // A
    idx = idx_flat.reshape(O, A)
    w = jax.nn.softmax(score[idx], axis=1)
    return jnp.sum(w[:, :, None] * table[idx], axis=1)


def kernel(node_embs, edge_embs, edge_list, node_list, W1, W2, a1, a2):
    a1_col = a1
    a2_col = a2[:D]
    Wh, s1 = _mm_score(node_embs, W1, a1_col, leaky=True)
    new_edge_embs = _agg(Wh, s1.reshape(-1), edge_list.reshape(-1), AN)
    Wf_packed, s2 = _mm_score(new_edge_embs, W2, a2_col, leaky=False,
                              pack=True)
    new_node_embs = _agg(Wf_packed, s2.reshape(-1), node_list.reshape(-1), AE,
                         packed_bf16=True)
    return (new_node_embs, new_edge_embs)


# speedup vs baseline: 1.0330x; 1.0330x over previous
import functools

import jax
import jax.numpy as jnp
from jax import lax
from jax.experimental import pallas as pl
from jax.experimental.pallas import tpu as pltpu
from jax.experimental.pallas import tpu_sc as plsc

N = 10000
M = 10000
D = 128
AN = 8
AE = 16
ALPHA = 0.2

L = 16
NW = 32
NB = 10000 // L
LSTAGE = 20
assert (NW - 1) * NB // NW + LSTAGE <= NB


def _mm_score_body(x_ref, w_ref, a_ref, y_ref, s_ref, *, leaky, pack):
    y = jnp.dot(x_ref[...], w_ref[...], preferred_element_type=jnp.float32)
    if pack:
        yi = jax.lax.bitcast_convert_type(y, jnp.int32) + 0x8000
        lo = jnp.right_shift(yi[:, : D // 2], 16) & 0xFFFF
        hi = yi[:, D // 2:] & jnp.int32(-65536)
        y_ref[...] = lo | hi
    else:
        y_ref[...] = y
    u = jnp.where(y >= 0, y, ALPHA * y) if leaky else y
    s_ref[...] = jnp.dot(u, a_ref[...], preferred_element_type=jnp.float32)


def _mm_score(x, w, a_row, leaky, pack=False):
    R = x.shape[0]
    B = 2000
    YW = D // 2 if pack else D
    return pl.pallas_call(
        functools.partial(_mm_score_body, leaky=leaky, pack=pack),
        grid=(R // B,),
        in_specs=[
            pl.BlockSpec((B, D), lambda i: (i, 0)),
            pl.BlockSpec((D, D), lambda i: (0, 0)),
            pl.BlockSpec((D, 1), lambda i: (0, 0)),
        ],
        out_specs=[
            pl.BlockSpec((B, YW), lambda i: (i, 0)),
            pl.BlockSpec((B, 1), lambda i: (i, 0)),
        ],
        out_shape=[
            jax.ShapeDtypeStruct((R, YW), jnp.int32 if pack else jnp.float32),
            jax.ShapeDtypeStruct((R, 1), jnp.float32),
        ],
    )(x, w, a_row)


def _agg_body(table_hbm, score_hbm, idx_hbm, out_hbm,
              score_v, el_v, rows, w_v, o_v, sems, semo, sem_s,
              *, A, T, packed_bf16):
    IPB = L * A
    c = lax.axis_index("c")
    s = lax.axis_index("s")
    wid = s * 2 + c
    b0 = wid * NB // NW
    b1 = (wid + 1) * NB // NW
    nb = b1 - b0

    score_cp = pltpu.make_async_copy(score_hbm, score_v, sem_s)
    score_cp.start()
    pltpu.sync_copy(idx_hbm.at[pl.ds(b0 * IPB, LSTAGE * IPB)], el_v)
    lanes = lax.iota(jnp.int32, L)

    def start(i, rows, sem):
        pltpu.make_async_copy(
            table_hbm.at[el_v.at[pl.ds(i * IPB, IPB)]], rows, sem).start()

    def softmax_weights(i):
        off = i * IPB
        sjs = []
        for j in range(A):
            ij = plsc.load_gather(el_v, [off + lanes * A + j])
            sjs.append(plsc.load_gather(score_v, [ij]))
        mx = sjs[0]
        for j in range(1, A):
            mx = jnp.maximum(mx, sjs[j])
        es = [jnp.exp(sj - mx) for sj in sjs]
        z = es[0]
        for j in range(1, A):
            z = z + es[j]
        rz = 1.0 / z
        for j in range(A):
            wv = es[j] * rz
            if packed_bf16:
                w_v[pl.ds(j * L, L)] = plsc.bitcast(
                    plsc.pack(wv, wv, format=plsc.PackFormat.INTERLEAVED),
                    jnp.int32)
            else:
                w_v[pl.ds(j * L, L)] = wv

    def weighted_sum(i, rows_v, o_v):
        def edge_body(e, carry2):
            acc = [jnp.zeros((L,), jnp.float32)] * (D // L)
            for j in range(A):
                wj = plsc.load_gather(
                    w_v, [jnp.broadcast_to(j * L + e, (L,))])
                r = e * A + j
                if not packed_bf16:
                    for ch in range(D // L):
                        acc[ch] = acc[ch] + wj * rows_v[r, pl.ds(ch * L, L)]
                else:
                    wb = plsc.bitcast(wj, jnp.bfloat16)
                    nh = D // (2 * L)
                    for ch in range(nh):
                        word = rows_v[r, pl.ds(ch * L, L)]
                        prod = plsc.bitcast(word, jnp.bfloat16) * wb
                        ev, od = plsc.unpack(
                            prod, format=plsc.PackFormat.INTERLEAVED)
                        acc[ch] = acc[ch] + ev
                        acc[nh + ch] = acc[nh + ch] + od
            for ch in range(D // L):
                o_v[e, pl.ds(ch * L, L)] = acc[ch]
            return carry2

        lax.fori_loop(0, L, edge_body, 0)

    def out_start(i, o_v, semo):
        pltpu.make_async_copy(
            o_v, out_hbm.at[pl.ds((b0 + i) * L, L)], semo).start()

    def out_wait(i, o_v, semo):
        pltpu.make_async_copy(
            o_v, out_hbm.at[pl.ds((b0 + i) * L, L)], semo).wait()

    start(0, rows[0], sems[0])
    start(1, rows[1], sems[1])
    start(2, rows[2], sems[2])
    score_cp.wait()

    def stage(g, p, k):
        @pl.when(g + 3 < nb)
        def _():
            start(g + 3, rows[(k + 3) % 4], sems[(k + 3) % 4])

        softmax_weights(g)
        if k < 2:
            @pl.when(p > 0)
            def _():
                out_wait(g - 2, o_v[k % 2], semo[k % 2])
        else:
            out_wait(g - 2, o_v[k % 2], semo[k % 2])
        pltpu.make_async_copy(
            table_hbm.at[el_v.at[pl.ds(g * IPB, IPB)]],
            rows[k], sems[k]).wait()
        weighted_sum(g, rows[k], o_v[k % 2])
        out_start(g, o_v[k % 2], semo[k % 2])

    def quad_body(p, carry):
        g = 4 * p
        stage(g, p, 0)
        stage(g + 1, p, 1)
        stage(g + 2, p, 2)

        @pl.when(g + 3 < nb)
        def _():
            stage(g + 3, p, 3)

        return carry

    lax.fori_loop(0, LSTAGE // 4, quad_body, 0)
    out_wait(LSTAGE - 2, o_v[0], semo[0])
    out_wait(LSTAGE - 2, o_v[1], semo[1])


def _agg(table, score, idx_flat, A, packed_bf16=False):
    O = idx_flat.shape[0] // A
    T = table.shape[0]
    W = table.shape[1]
    mesh = plsc.VectorSubcoreMesh(core_axis_name="c", subcore_axis_name="s")
    kfn = functools.partial(
        pl.kernel,
        out_type=jax.ShapeDtypeStruct((O, D), jnp.float32),
        mesh=mesh,
        compiler_params=pltpu.CompilerParams(
            needs_layout_passes=False,
            use_tc_tiling_on_sc=not packed_bf16),
        scratch_types=[
            pltpu.VMEM((T,), jnp.float32),
            pltpu.VMEM((LSTAGE * L * A,), jnp.int32),
            [pltpu.VMEM((L * A, W), table.dtype)
             for _ in range(4)],
            pltpu.VMEM((A * L,),
                       jnp.int32 if packed_bf16 else jnp.float32),
            [pltpu.VMEM((L, D), jnp.float32)
             for _ in range(2)],
            [pltpu.SemaphoreType.DMA for _ in range(4)],
            [pltpu.SemaphoreType.DMA for _ in range(2)],
            pltpu.SemaphoreType.DMA,
        ],
    )
    body = functools.partial(_agg_body, A=A, T=T, packed_bf16=packed_bf16)
    return kfn(body)(table, score, idx_flat)


def _agg_jnp(table, score, idx_flat, A):
    O = idx_flat.shape[0] // A
    idx = idx_flat.reshape(O, A)
    w = jax.nn.softmax(score[idx], axis=1)
    return jnp.sum(w[:, :, None] * table[idx], axis=1)


def kernel(node_embs, edge_embs, edge_list, node_list, W1, W2, a1, a2):
    a1_col = a1
    a2_col = a2[:D]
    Wh, s1 = _mm_score(node_embs, W1, a1_col, leaky=True)
    new_edge_embs = _agg(Wh, s1.reshape(-1), edge_list.reshape(-1), AN)
    Wf_packed, s2 = _mm_score(new_edge_embs, W2, a2_col, leaky=False,
                              pack=True)
    new_node_embs = _agg(Wf_packed, s2.reshape(-1), node_list.reshape(-1), AE,
                         packed_bf16=True)
    return (new_node_embs, new_edge_embs)
